# baseline (speedup 1.0000x reference)
import jax
import jax.numpy as jnp
import numpy as np
from jax.experimental import pallas as pl
from jax.experimental.pallas import tpu as pltpu

_T = 512


def _tile_kernel(ij_ref, xiT_ref, xjT_ref, out_ref, acc_ref):
    t = pl.program_id(0)
    is_compute = ij_ref[2, t] == 1

    @pl.when(is_compute)
    def _compute():
        xjT = xjT_ref[:, :]
        for a in range(_T):
            col = xiT_ref[:, a : a + 1]
            acc_ref[a : a + 1, :] = jnp.max(
                jnp.abs(xjT - col), axis=0, keepdims=True
            )
        out_ref[:, :] = acc_ref[:, :]

    @pl.when(jnp.logical_not(is_compute))
    def _mirror():
        out_ref[:, :] = acc_ref[:, :].T


def _pairwise_inf(xT, steps, n, d, interpret=False):
    nsteps = steps.shape[1]
    grid_spec = pltpu.PrefetchScalarGridSpec(
        num_scalar_prefetch=1,
        grid=(nsteps,),
        in_specs=[
            pl.BlockSpec((d, _T), lambda t, ij: (0, ij[0, t])),
            pl.BlockSpec((d, _T), lambda t, ij: (0, ij[1, t])),
        ],
        out_specs=pl.BlockSpec((_T, _T), lambda t, ij: (ij[3, t], ij[4, t])),
        scratch_shapes=[pltpu.VMEM((_T, _T), xT.dtype)],
    )
    return pl.pallas_call(
        _tile_kernel,
        grid_spec=grid_spec,
        out_shape=jax.ShapeDtypeStruct((n, n), xT.dtype),
        interpret=interpret,
    )(steps, xT, xT)


def _make_steps(nb):
    cols = []
    for i in range(nb):
        cols.append((i, i, 1, i, i))
        for j in range(i + 1, nb):
            cols.append((i, j, 1, i, j))
            cols.append((i, j, 0, j, i))
    return np.array(cols, dtype=np.int32).T


def kernel(x):
    n, d = x.shape
    steps = _make_steps(n // _T)
    return _pairwise_inf(x.T, jnp.asarray(steps), n, d)


# speedup vs baseline: 1.3985x; 1.3985x over previous
import jax
import jax.numpy as jnp
import numpy as np
from jax.experimental import pallas as pl
from jax.experimental.pallas import tpu as pltpu

_TI = 256
_TJ = 512


def _tile_kernel(ij_ref, xiT_ref, xjT_ref, out_u_ref, out_l_ref):
    xjT = xjT_ref[:, :]
    for a in range(_TI):
        col = xiT_ref[:, a : a + 1]
        out_u_ref[a : a + 1, :] = jnp.max(
            jnp.abs(xjT - col), axis=0, keepdims=True
        )
    out_l_ref[:, :] = out_u_ref[:, :].T


def _pairwise_inf(xT, steps, n, d, interpret=False):
    nsteps = steps.shape[1]
    grid_spec = pltpu.PrefetchScalarGridSpec(
        num_scalar_prefetch=1,
        grid=(nsteps,),
        in_specs=[
            pl.BlockSpec((d, _TI), lambda t, ij: (0, ij[0, t])),
            pl.BlockSpec((d, _TJ), lambda t, ij: (0, ij[1, t])),
        ],
        out_specs=[
            pl.BlockSpec((_TI, _TJ), lambda t, ij: (ij[0, t], ij[1, t])),
            pl.BlockSpec((_TJ, _TI), lambda t, ij: (ij[1, t], ij[0, t])),
        ],
    )
    out_u, out_l = pl.pallas_call(
        _tile_kernel,
        grid_spec=grid_spec,
        out_shape=[
            jax.ShapeDtypeStruct((n, n), xT.dtype),
            jax.ShapeDtypeStruct((n, n), xT.dtype),
        ],
        interpret=interpret,
    )(steps, xT, xT)
    r = jax.lax.broadcasted_iota(jnp.int32, (n, n), 0)
    c = jax.lax.broadcasted_iota(jnp.int32, (n, n), 1)
    return jnp.where(c >= r, out_u, out_l)


def _make_steps(n):
    nbi = n // _TI
    nbj = n // _TJ
    cols = []
    for i in range(nbi):
        for jp in range(i // 2, nbj):
            cols.append((i, jp))
    return np.array(cols, dtype=np.int32).T


def kernel(x):
    n, d = x.shape
    steps = _make_steps(n)
    return _pairwise_inf(x.T, jnp.asarray(steps), n, d)
